# baseline (speedup 1.0000x reference)
import functools

import jax
import jax.numpy as jnp
from jax import lax
from jax.experimental import pallas as pl
from jax.experimental.pallas import tpu as pltpu
from jax.experimental.pallas import tpu_sc as plsc

N = 10000
E = 320000
NFEAT = 128
NHID = 128
NCLASS = 40
NLAYER = 6

_NC = 2
_NS = 16
_NW = _NC * _NS
_EPT = E // _NW
_CH = 125
_NJ = _EPT // _CH
_HF = _NJ // 2
_RB = 624
_RL = 640

_DCH = 100
_DNJ = _EPT // _DCH

_TCBLK = 1000


def _zero_vmem(ref, rows, cols):
    @pl.loop(0, rows)
    def _(r):
        @pl.loop(0, cols // 16)
        def _(c):
            ref[r, pl.ds(c * 16, 16)] = jnp.zeros((16,), jnp.float32)


def _sc_degree(dst3):
    mesh = plsc.VectorSubcoreMesh(core_axis_name="c", subcore_axis_name="s")

    @functools.partial(
        pl.kernel,
        out_type=jax.ShapeDtypeStruct((_NC * N, 16), jnp.float32),
        mesh=mesh,
        scratch_types=[
            pltpu.VMEM_SHARED((N, 16), jnp.float32),
            pltpu.VMEM((_DNJ, _DCH), jnp.int32),
            pltpu.VMEM((_DCH, 16), jnp.float32),
            pltpu.VMEM((128, 16), jnp.float32),
        ],
    )
    def k(dst_hbm, out_hbm, hist_sh, dstv, ones_v, zbuf):
        cid = lax.axis_index("c")
        sid = lax.axis_index("s")
        wid = cid * _NS + sid
        @pl.loop(0, _DCH)
        def _(r):
            ones_v[r, pl.ds(0, 16)] = jnp.ones((16,), jnp.float32)
        _zero_vmem(zbuf, 128, 16)
        for t in range(5):
            pltpu.sync_copy(zbuf, hist_sh.at[pl.ds(sid * _RB + t * 128, 128)])
        plsc.subcore_barrier()
        pltpu.sync_copy(dst_hbm.at[wid], dstv)
        @pl.loop(0, _DNJ)
        def _(j):
            pltpu.sync_copy(ones_v, hist_sh.at[dstv.at[j]], add=True)
        plsc.subcore_barrier()
        pltpu.sync_copy(
            hist_sh.at[pl.ds(sid * _RB, _RL)],
            out_hbm.at[pl.ds(cid * N + sid * _RB, _RL)],
        )

    return k(dst3)


def _sc_spmm(y, src3, dst3):
    mesh = plsc.VectorSubcoreMesh(core_axis_name="c", subcore_axis_name="s")

    @functools.partial(
        pl.kernel,
        out_type=jax.ShapeDtypeStruct((_NC * N, NHID), jnp.float32),
        mesh=mesh,
        scratch_types=[
            pltpu.VMEM_SHARED((N, NHID), jnp.float32),
            pltpu.VMEM((_HF, _CH), jnp.int32),
            pltpu.VMEM((_HF, _CH), jnp.int32),
            pltpu.VMEM((_CH, NHID), jnp.float32),
            pltpu.VMEM((_CH, NHID), jnp.float32),
            pltpu.SemaphoreType.DMA,
            pltpu.SemaphoreType.DMA,
            pltpu.SemaphoreType.DMA,
            pltpu.SemaphoreType.DMA,
        ],
    )
    def k(y_hbm, src_hbm, dst_hbm, out_hbm, acc_sh, srcv, dstv, rows0, rows1,
          sem0, sem1, sems0, sems1):
        cid = lax.axis_index("c")
        sid = lax.axis_index("s")
        wid = cid * _NS + sid
        _zero_vmem(rows0, 64, NHID)
        for t in range(10):
            pltpu.sync_copy(rows0.at[pl.ds(0, 64)],
                            acc_sh.at[pl.ds(sid * _RB + t * 64, 64)])
        plsc.subcore_barrier()
        pltpu.sync_copy(src_hbm.at[wid, pl.ds(0, _HF)], srcv)
        pltpu.sync_copy(dst_hbm.at[wid, pl.ds(0, _HF)], dstv)
        pltpu.async_copy(y_hbm.at[srcv.at[0]], rows0, sem0)
        pltpu.async_copy(y_hbm.at[srcv.at[1]], rows1, sem1)
        for h in range(2):
            @pl.loop(0, _HF, step=2)
            def _(j):
                pltpu.make_async_copy(y_hbm.at[srcv.at[j]], rows0, sem0).wait()
                d0 = pltpu.async_copy(rows0, acc_sh.at[dstv.at[j]], sems0,
                                      add=True)
                pltpu.make_async_copy(y_hbm.at[srcv.at[j + 1]], rows1,
                                      sem1).wait()
                d1 = pltpu.async_copy(rows1, acc_sh.at[dstv.at[j + 1]], sems1,
                                      add=True)
                d0.wait()
                @pl.when(j + 2 < _HF)
                def _():
                    pltpu.async_copy(y_hbm.at[srcv.at[j + 2]], rows0, sem0)
                d1.wait()
                @pl.when(j + 3 < _HF)
                def _():
                    pltpu.async_copy(y_hbm.at[srcv.at[j + 3]], rows1, sem1)
            if h == 0:
                pltpu.sync_copy(src_hbm.at[wid, pl.ds(_HF, _HF)], srcv)
                pltpu.sync_copy(dst_hbm.at[wid, pl.ds(_HF, _HF)], dstv)
                pltpu.async_copy(y_hbm.at[srcv.at[0]], rows0, sem0)
                pltpu.async_copy(y_hbm.at[srcv.at[1]], rows1, sem1)
        plsc.subcore_barrier()
        pltpu.sync_copy(
            acc_sh.at[pl.ds(sid * _RB, _RL)],
            out_hbm.at[pl.ds(cid * N + sid * _RB, _RL)],
        )

    return k(y, src3, dst3)


def _dinv_block(deg2_blk):
    deg = deg2_blk[0, :, 0] + deg2_blk[1, :, 0] + 1.0
    return lax.rsqrt(deg)[:, None]


def _tc_pre_body(deg2_ref, x_ref, w_ref, y_ref):
    dinv = _dinv_block(deg2_ref[...])
    y_ref[...] = dinv * jnp.dot(x_ref[...], w_ref[...],
                                preferred_element_type=jnp.float32)


def _tc_pre(deg2, x, w):
    return pl.pallas_call(
        _tc_pre_body,
        grid=(N // _TCBLK,),
        in_specs=[
            pl.BlockSpec((_NC, _TCBLK, 16), lambda i: (0, i, 0)),
            pl.BlockSpec((_TCBLK, NFEAT), lambda i: (i, 0)),
            pl.BlockSpec((NFEAT, NHID), lambda i: (0, 0)),
        ],
        out_specs=pl.BlockSpec((_TCBLK, NHID), lambda i: (i, 0)),
        out_shape=jax.ShapeDtypeStruct((N, NHID), jnp.float32),
    )(deg2, x, w)


def _tc_mid_body(deg2_ref, acc2_ref, y_ref, b_ref, w_ref, *rest, first):
    if first:
        ynext_ref, mout_ref = rest
    else:
        m_ref, ynext_ref, mout_ref = rest
    dinv = _dinv_block(deg2_ref[...])
    agg = acc2_ref[0] + acc2_ref[1] + y_ref[...]
    h = jnp.maximum(dinv * agg + b_ref[...], 0.0)
    m = h if first else jnp.maximum(m_ref[...], h)
    mout_ref[...] = m
    ynext_ref[...] = dinv * jnp.dot(h, w_ref[...],
                                    preferred_element_type=jnp.float32)


def _tc_mid(deg2, acc2, y, b, w, m):
    first = m is None
    blk = pl.BlockSpec((_TCBLK, NHID), lambda i: (i, 0))
    in_specs = [
        pl.BlockSpec((_NC, _TCBLK, 16), lambda i: (0, i, 0)),
        pl.BlockSpec((_NC, _TCBLK, NHID), lambda i: (0, i, 0)),
        blk,
        pl.BlockSpec((1, NHID), lambda i: (0, 0)),
        pl.BlockSpec((NHID, NHID), lambda i: (0, 0)),
    ]
    args = [deg2, acc2, y, b, w]
    if not first:
        in_specs.append(blk)
        args.append(m)
    return pl.pallas_call(
        functools.partial(_tc_mid_body, first=first),
        grid=(N // _TCBLK,),
        in_specs=in_specs,
        out_specs=[blk, blk],
        out_shape=[
            jax.ShapeDtypeStruct((N, NHID), jnp.float32),
            jax.ShapeDtypeStruct((N, NHID), jnp.float32),
        ],
    )(*args)


def _tc_post_body(deg2_ref, acc2_ref, y_ref, b_ref, m_ref, fcw_ref, fcb_ref,
                  out_ref):
    dinv = _dinv_block(deg2_ref[...])
    agg = acc2_ref[0] + acc2_ref[1] + y_ref[...]
    h = jnp.maximum(dinv * agg + b_ref[...], 0.0)
    m = jnp.maximum(m_ref[...], h)
    lg = jnp.dot(m, fcw_ref[...], preferred_element_type=jnp.float32)
    lg = lg + fcb_ref[...]
    mx = jnp.max(lg, axis=1, keepdims=True)
    out_ref[...] = (lg - mx) - jnp.log(
        jnp.sum(jnp.exp(lg - mx), axis=1, keepdims=True))


def _tc_post(deg2, acc2, y, b, m, fc_w, fc_b):
    blk = pl.BlockSpec((_TCBLK, NHID), lambda i: (i, 0))
    return pl.pallas_call(
        _tc_post_body,
        grid=(N // _TCBLK,),
        in_specs=[
            pl.BlockSpec((_NC, _TCBLK, 16), lambda i: (0, i, 0)),
            pl.BlockSpec((_NC, _TCBLK, NHID), lambda i: (0, i, 0)),
            blk,
            pl.BlockSpec((1, NHID), lambda i: (0, 0)),
            blk,
            pl.BlockSpec((NHID, NCLASS), lambda i: (0, 0)),
            pl.BlockSpec((1, NCLASS), lambda i: (0, 0)),
        ],
        out_specs=pl.BlockSpec((_TCBLK, NCLASS), lambda i: (i, 0)),
        out_shape=jax.ShapeDtypeStruct((N, NCLASS), jnp.float32),
    )(deg2, acc2, y, b, m, fc_w, fc_b)


def kernel(x, edge_index, W0, b0, W1, b1, W2, b2, W3, b3, W4, b4, W5, b5,
           fc_W, fc_b):
    Ws = [W0, W1, W2, W3, W4, W5]
    bs = [b.reshape(1, NHID) for b in (b0, b1, b2, b3, b4, b5)]
    src3 = edge_index[0].reshape(_NW, _NJ, _CH)
    dst3s = edge_index[1].reshape(_NW, _NJ, _CH)
    dst3 = edge_index[1].reshape(_NW, _DNJ, _DCH)

    deg2 = _sc_degree(dst3).reshape(_NC, N, 16)
    y = _tc_pre(deg2, x, Ws[0])
    m = None
    for i in range(NLAYER):
        acc2 = _sc_spmm(y, src3, dst3s).reshape(_NC, N, NHID)
        if i < NLAYER - 1:
            y, m = _tc_mid(deg2, acc2, y, bs[i], Ws[i + 1], m)
        else:
            out = _tc_post(deg2, acc2, y, bs[i], m, fc_W,
                           fc_b.reshape(1, NCLASS))
    return out


# speedup vs baseline: 1.2617x; 1.2617x over previous
import functools

import jax
import jax.numpy as jnp
from jax import lax
from jax.experimental import pallas as pl
from jax.experimental.pallas import tpu as pltpu
from jax.experimental.pallas import tpu_sc as plsc

N = 10000
E = 320000
NFEAT = 128
NHID = 128
NCLASS = 40
NLAYER = 6

_NC = 2
_NS = 16
_NW = _NC * _NS
_EPT = E // _NW
_CH = 125
_NJ = _EPT // _CH
_HF = _NJ // 2
_RB = 624
_RL = 640

_DCH = _CH
_DNJ = _EPT // _DCH

_TCBLK = 1000


def _zero_vmem(ref, rows, cols):
    @pl.loop(0, rows)
    def _(r):
        @pl.loop(0, cols // 16)
        def _(c):
            ref[r, pl.ds(c * 16, 16)] = jnp.zeros((16,), jnp.float32)


def _sc_degree(dst3):
    mesh = plsc.VectorSubcoreMesh(core_axis_name="c", subcore_axis_name="s")

    @functools.partial(
        pl.kernel,
        out_type=jax.ShapeDtypeStruct((_NC * N, 16), jnp.float32),
        mesh=mesh,
        scratch_types=[
            pltpu.VMEM_SHARED((N, 16), jnp.float32),
            pltpu.VMEM((_DNJ, _DCH), jnp.int32),
            pltpu.VMEM((_DCH, 16), jnp.float32),
            pltpu.VMEM((128, 16), jnp.float32),
        ],
    )
    def k(dst_hbm, out_hbm, hist_sh, dstv, ones_v, zbuf):
        cid = lax.axis_index("c")
        sid = lax.axis_index("s")
        wid = cid * _NS + sid
        @pl.loop(0, _DCH)
        def _(r):
            ones_v[r, pl.ds(0, 16)] = jnp.ones((16,), jnp.float32)
        _zero_vmem(zbuf, 128, 16)
        for t in range(5):
            pltpu.sync_copy(zbuf, hist_sh.at[pl.ds(sid * _RB + t * 128, 128)])
        plsc.subcore_barrier()
        pltpu.sync_copy(dst_hbm.at[wid], dstv)
        @pl.loop(0, _DNJ)
        def _(j):
            pltpu.sync_copy(ones_v, hist_sh.at[dstv.at[j]], add=True)
        plsc.subcore_barrier()
        pltpu.sync_copy(
            hist_sh.at[pl.ds(sid * _RB, _RL)],
            out_hbm.at[pl.ds(cid * N + sid * _RB, _RL)],
        )

    return k(dst3)


def _sc_spmm(y, src3, dst3):
    mesh = plsc.VectorSubcoreMesh(core_axis_name="c", subcore_axis_name="s")

    @functools.partial(
        pl.kernel,
        out_type=jax.ShapeDtypeStruct((_NC * N, NHID), jnp.float32),
        mesh=mesh,
        scratch_types=[
            pltpu.VMEM_SHARED((N, NHID), jnp.float32),
            pltpu.VMEM((_HF, _CH), jnp.int32),
            pltpu.VMEM((_HF, _CH), jnp.int32),
            pltpu.VMEM((_CH, NHID), jnp.float32),
            pltpu.VMEM((_CH, NHID), jnp.float32),
            pltpu.SemaphoreType.DMA,
            pltpu.SemaphoreType.DMA,
        ],
    )
    def k(y_hbm, src_hbm, dst_hbm, out_hbm, acc_sh, srcv, dstv, rows0, rows1,
          sem0, sem1):
        cid = lax.axis_index("c")
        sid = lax.axis_index("s")
        wid = cid * _NS + sid
        _zero_vmem(rows0, 64, NHID)
        for t in range(10):
            pltpu.sync_copy(rows0.at[pl.ds(0, 64)],
                            acc_sh.at[pl.ds(sid * _RB + t * 64, 64)])
        plsc.subcore_barrier()
        pltpu.sync_copy(src_hbm.at[wid, pl.ds(0, _HF)], srcv)
        pltpu.sync_copy(dst_hbm.at[wid, pl.ds(0, _HF)], dstv)
        pltpu.async_copy(y_hbm.at[srcv.at[0]], rows0, sem0)
        for h in range(2):
            @pl.loop(0, _HF, step=2)
            def _(j):
                pltpu.async_copy(y_hbm.at[srcv.at[j + 1]], rows1, sem1)
                pltpu.make_async_copy(y_hbm.at[srcv.at[j]], rows0, sem0).wait()
                pltpu.sync_copy(rows0, acc_sh.at[dstv.at[j]], add=True)
                @pl.when(j + 2 < _HF)
                def _():
                    pltpu.async_copy(y_hbm.at[srcv.at[j + 2]], rows0, sem0)
                pltpu.make_async_copy(y_hbm.at[srcv.at[j + 1]], rows1,
                                      sem1).wait()
                pltpu.sync_copy(rows1, acc_sh.at[dstv.at[j + 1]], add=True)
            if h == 0:
                pltpu.sync_copy(src_hbm.at[wid, pl.ds(_HF, _HF)], srcv)
                pltpu.sync_copy(dst_hbm.at[wid, pl.ds(_HF, _HF)], dstv)
                pltpu.async_copy(y_hbm.at[srcv.at[0]], rows0, sem0)
        plsc.subcore_barrier()
        pltpu.sync_copy(
            acc_sh.at[pl.ds(sid * _RB, _RL)],
            out_hbm.at[pl.ds(cid * N + sid * _RB, _RL)],
        )

    return k(y, src3, dst3)


def _dinv_block(deg2_blk):
    deg = deg2_blk[0, :, 0] + deg2_blk[1, :, 0] + 1.0
    return lax.rsqrt(deg)[:, None]


def _tc_pre_body(deg2_ref, x_ref, w_ref, y_ref):
    dinv = _dinv_block(deg2_ref[...])
    y_ref[...] = dinv * jnp.dot(x_ref[...], w_ref[...],
                                preferred_element_type=jnp.float32)


def _tc_pre(deg2, x, w):
    return pl.pallas_call(
        _tc_pre_body,
        grid=(N // _TCBLK,),
        in_specs=[
            pl.BlockSpec((_NC, _TCBLK, 16), lambda i: (0, i, 0)),
            pl.BlockSpec((_TCBLK, NFEAT), lambda i: (i, 0)),
            pl.BlockSpec((NFEAT, NHID), lambda i: (0, 0)),
        ],
        out_specs=pl.BlockSpec((_TCBLK, NHID), lambda i: (i, 0)),
        out_shape=jax.ShapeDtypeStruct((N, NHID), jnp.float32),
    )(deg2, x, w)


def _tc_mid_body(deg2_ref, acc2_ref, y_ref, b_ref, w_ref, *rest, first):
    if first:
        ynext_ref, mout_ref = rest
    else:
        m_ref, ynext_ref, mout_ref = rest
    dinv = _dinv_block(deg2_ref[...])
    agg = acc2_ref[0] + acc2_ref[1] + y_ref[...]
    h = jnp.maximum(dinv * agg + b_ref[...], 0.0)
    m = h if first else jnp.maximum(m_ref[...], h)
    mout_ref[...] = m
    ynext_ref[...] = dinv * jnp.dot(h, w_ref[...],
                                    preferred_element_type=jnp.float32)


def _tc_mid(deg2, acc2, y, b, w, m):
    first = m is None
    blk = pl.BlockSpec((_TCBLK, NHID), lambda i: (i, 0))
    in_specs = [
        pl.BlockSpec((_NC, _TCBLK, 16), lambda i: (0, i, 0)),
        pl.BlockSpec((_NC, _TCBLK, NHID), lambda i: (0, i, 0)),
        blk,
        pl.BlockSpec((1, NHID), lambda i: (0, 0)),
        pl.BlockSpec((NHID, NHID), lambda i: (0, 0)),
    ]
    args = [deg2, acc2, y, b, w]
    if not first:
        in_specs.append(blk)
        args.append(m)
    return pl.pallas_call(
        functools.partial(_tc_mid_body, first=first),
        grid=(N // _TCBLK,),
        in_specs=in_specs,
        out_specs=[blk, blk],
        out_shape=[
            jax.ShapeDtypeStruct((N, NHID), jnp.float32),
            jax.ShapeDtypeStruct((N, NHID), jnp.float32),
        ],
    )(*args)


def _tc_post_body(deg2_ref, acc2_ref, y_ref, b_ref, m_ref, fcw_ref, fcb_ref,
                  out_ref):
    dinv = _dinv_block(deg2_ref[...])
    agg = acc2_ref[0] + acc2_ref[1] + y_ref[...]
    h = jnp.maximum(dinv * agg + b_ref[...], 0.0)
    m = jnp.maximum(m_ref[...], h)
    lg = jnp.dot(m, fcw_ref[...], preferred_element_type=jnp.float32)
    lg = lg + fcb_ref[...]
    mx = jnp.max(lg, axis=1, keepdims=True)
    out_ref[...] = (lg - mx) - jnp.log(
        jnp.sum(jnp.exp(lg - mx), axis=1, keepdims=True))


def _tc_post(deg2, acc2, y, b, m, fc_w, fc_b):
    blk = pl.BlockSpec((_TCBLK, NHID), lambda i: (i, 0))
    return pl.pallas_call(
        _tc_post_body,
        grid=(N // _TCBLK,),
        in_specs=[
            pl.BlockSpec((_NC, _TCBLK, 16), lambda i: (0, i, 0)),
            pl.BlockSpec((_NC, _TCBLK, NHID), lambda i: (0, i, 0)),
            blk,
            pl.BlockSpec((1, NHID), lambda i: (0, 0)),
            blk,
            pl.BlockSpec((NHID, NCLASS), lambda i: (0, 0)),
            pl.BlockSpec((1, NCLASS), lambda i: (0, 0)),
        ],
        out_specs=pl.BlockSpec((_TCBLK, NCLASS), lambda i: (i, 0)),
        out_shape=jax.ShapeDtypeStruct((N, NCLASS), jnp.float32),
    )(deg2, acc2, y, b, m, fc_w, fc_b)


def kernel(x, edge_index, W0, b0, W1, b1, W2, b2, W3, b3, W4, b4, W5, b5,
           fc_W, fc_b):
    Ws = [W0, W1, W2, W3, W4, W5]
    bs = [b.reshape(1, NHID) for b in (b0, b1, b2, b3, b4, b5)]
    src3 = edge_index[0].reshape(_NW, _NJ, _CH)
    dst3s = edge_index[1].reshape(_NW, _NJ, _CH)

    deg2 = _sc_degree(dst3s).reshape(_NC, N, 16)
    y = _tc_pre(deg2, x, Ws[0])
    m = None
    for i in range(NLAYER):
        acc2 = _sc_spmm(y, src3, dst3s).reshape(_NC, N, NHID)
        if i < NLAYER - 1:
            y, m = _tc_mid(deg2, acc2, y, bs[i], Ws[i + 1], m)
        else:
            out = _tc_post(deg2, acc2, y, bs[i], m, fc_W,
                           fc_b.reshape(1, NCLASS))
    return out


# speedup vs baseline: 1.2636x; 1.0014x over previous
import functools

import jax
import jax.numpy as jnp
from jax import lax
from jax.experimental import pallas as pl
from jax.experimental.pallas import tpu as pltpu
from jax.experimental.pallas import tpu_sc as plsc

N = 10000
E = 320000
NFEAT = 128
NHID = 128
NCLASS = 40
NLAYER = 6

_NC = 2
_NS = 16
_NW = _NC * _NS
_EPT = E // _NW
_CH = 125
_NJ = _EPT // _CH
_HF = _NJ // 2
_RB = 624
_RL = 640

_DCH = 100
_DNJ = _EPT // _DCH

_TCBLK = 1000


def _zero_vmem(ref, rows, cols):
    @pl.loop(0, rows)
    def _(r):
        @pl.loop(0, cols // 16)
        def _(c):
            ref[r, pl.ds(c * 16, 16)] = jnp.zeros((16,), jnp.float32)


def _sc_degree(dst3):
    mesh = plsc.VectorSubcoreMesh(core_axis_name="c", subcore_axis_name="s")

    @functools.partial(
        pl.kernel,
        out_type=jax.ShapeDtypeStruct((_NC * N, 16), jnp.float32),
        mesh=mesh,
        scratch_types=[
            pltpu.VMEM_SHARED((N, 16), jnp.float32),
            pltpu.VMEM((_DNJ, _DCH), jnp.int32),
            pltpu.VMEM((_DCH, 16), jnp.float32),
            pltpu.VMEM((128, 16), jnp.float32),
        ],
    )
    def k(dst_hbm, out_hbm, hist_sh, dstv, ones_v, zbuf):
        cid = lax.axis_index("c")
        sid = lax.axis_index("s")
        wid = cid * _NS + sid
        @pl.loop(0, _DCH)
        def _(r):
            ones_v[r, pl.ds(0, 16)] = jnp.ones((16,), jnp.float32)
        _zero_vmem(zbuf, 128, 16)
        for t in range(5):
            pltpu.sync_copy(zbuf, hist_sh.at[pl.ds(sid * _RB + t * 128, 128)])
        plsc.subcore_barrier()
        pltpu.sync_copy(dst_hbm.at[wid], dstv)
        @pl.loop(0, _DNJ)
        def _(j):
            pltpu.sync_copy(ones_v, hist_sh.at[dstv.at[j]], add=True)
        plsc.subcore_barrier()
        pltpu.sync_copy(
            hist_sh.at[pl.ds(sid * _RB, _RL)],
            out_hbm.at[pl.ds(cid * N + sid * _RB, _RL)],
        )

    return k(dst3)


def _sc_spmm(y, src3, dst3):
    mesh = plsc.VectorSubcoreMesh(core_axis_name="c", subcore_axis_name="s")

    @functools.partial(
        pl.kernel,
        out_type=jax.ShapeDtypeStruct((_NC * N, NHID), jnp.float32),
        mesh=mesh,
        scratch_types=[
            pltpu.VMEM_SHARED((N, NHID), jnp.float32),
            pltpu.VMEM((_HF, _CH), jnp.int32),
            pltpu.VMEM((_HF, _CH), jnp.int32),
            pltpu.VMEM((_CH, NHID), jnp.float32),
            pltpu.VMEM((_CH, NHID), jnp.float32),
            pltpu.SemaphoreType.DMA,
            pltpu.SemaphoreType.DMA,
        ],
    )
    def k(y_hbm, src_hbm, dst_hbm, out_hbm, acc_sh, srcv, dstv, rows0, rows1,
          sem0, sem1):
        cid = lax.axis_index("c")
        sid = lax.axis_index("s")
        wid = cid * _NS + sid
        _zero_vmem(rows0, 64, NHID)
        for t in range(10):
            pltpu.sync_copy(rows0.at[pl.ds(0, 64)],
                            acc_sh.at[pl.ds(sid * _RB + t * 64, 64)])
        plsc.subcore_barrier()
        pltpu.sync_copy(src_hbm.at[wid, pl.ds(0, _HF)], srcv)
        pltpu.sync_copy(dst_hbm.at[wid, pl.ds(0, _HF)], dstv)
        pltpu.async_copy(y_hbm.at[srcv.at[0]], rows0, sem0)
        for h in range(2):
            @pl.loop(0, _HF, step=2)
            def _(j):
                pltpu.async_copy(y_hbm.at[srcv.at[j + 1]], rows1, sem1)
                pltpu.make_async_copy(y_hbm.at[srcv.at[j]], rows0, sem0).wait()
                pltpu.sync_copy(rows0, acc_sh.at[dstv.at[j]], add=True)
                @pl.when(j + 2 < _HF)
                def _():
                    pltpu.async_copy(y_hbm.at[srcv.at[j + 2]], rows0, sem0)
                pltpu.make_async_copy(y_hbm.at[srcv.at[j + 1]], rows1,
                                      sem1).wait()
                pltpu.sync_copy(rows1, acc_sh.at[dstv.at[j + 1]], add=True)
            if h == 0:
                pltpu.sync_copy(src_hbm.at[wid, pl.ds(_HF, _HF)], srcv)
                pltpu.sync_copy(dst_hbm.at[wid, pl.ds(_HF, _HF)], dstv)
                pltpu.async_copy(y_hbm.at[srcv.at[0]], rows0, sem0)
        plsc.subcore_barrier()
        pltpu.sync_copy(
            acc_sh.at[pl.ds(sid * _RB, _RL)],
            out_hbm.at[pl.ds(cid * N + sid * _RB, _RL)],
        )

    return k(y, src3, dst3)


def _dinv_block(deg2_blk):
    deg = deg2_blk[0, :, 0] + deg2_blk[1, :, 0] + 1.0
    return lax.rsqrt(deg)[:, None]


def _tc_pre_body(deg2_ref, x_ref, w_ref, y_ref):
    dinv = _dinv_block(deg2_ref[...])
    y_ref[...] = dinv * jnp.dot(x_ref[...], w_ref[...],
                                preferred_element_type=jnp.float32)


def _tc_pre(deg2, x, w):
    return pl.pallas_call(
        _tc_pre_body,
        grid=(N // _TCBLK,),
        in_specs=[
            pl.BlockSpec((_NC, _TCBLK, 16), lambda i: (0, i, 0)),
            pl.BlockSpec((_TCBLK, NFEAT), lambda i: (i, 0)),
            pl.BlockSpec((NFEAT, NHID), lambda i: (0, 0)),
        ],
        out_specs=pl.BlockSpec((_TCBLK, NHID), lambda i: (i, 0)),
        out_shape=jax.ShapeDtypeStruct((N, NHID), jnp.float32),
    )(deg2, x, w)


def _tc_mid_body(deg2_ref, acc2_ref, y_ref, b_ref, w_ref, *rest, first):
    if first:
        ynext_ref, mout_ref = rest
    else:
        m_ref, ynext_ref, mout_ref = rest
    dinv = _dinv_block(deg2_ref[...])
    agg = acc2_ref[0] + acc2_ref[1] + y_ref[...]
    h = jnp.maximum(dinv * agg + b_ref[...], 0.0)
    m = h if first else jnp.maximum(m_ref[...], h)
    mout_ref[...] = m
    ynext_ref[...] = dinv * jnp.dot(h, w_ref[...],
                                    preferred_element_type=jnp.float32)


def _tc_mid(deg2, acc2, y, b, w, m):
    first = m is None
    blk = pl.BlockSpec((_TCBLK, NHID), lambda i: (i, 0))
    in_specs = [
        pl.BlockSpec((_NC, _TCBLK, 16), lambda i: (0, i, 0)),
        pl.BlockSpec((_NC, _TCBLK, NHID), lambda i: (0, i, 0)),
        blk,
        pl.BlockSpec((1, NHID), lambda i: (0, 0)),
        pl.BlockSpec((NHID, NHID), lambda i: (0, 0)),
    ]
    args = [deg2, acc2, y, b, w]
    if not first:
        in_specs.append(blk)
        args.append(m)
    return pl.pallas_call(
        functools.partial(_tc_mid_body, first=first),
        grid=(N // _TCBLK,),
        in_specs=in_specs,
        out_specs=[blk, blk],
        out_shape=[
            jax.ShapeDtypeStruct((N, NHID), jnp.float32),
            jax.ShapeDtypeStruct((N, NHID), jnp.float32),
        ],
    )(*args)


def _tc_post_body(deg2_ref, acc2_ref, y_ref, b_ref, m_ref, fcw_ref, fcb_ref,
                  out_ref):
    dinv = _dinv_block(deg2_ref[...])
    agg = acc2_ref[0] + acc2_ref[1] + y_ref[...]
    h = jnp.maximum(dinv * agg + b_ref[...], 0.0)
    m = jnp.maximum(m_ref[...], h)
    lg = jnp.dot(m, fcw_ref[...], preferred_element_type=jnp.float32)
    lg = lg + fcb_ref[...]
    mx = jnp.max(lg, axis=1, keepdims=True)
    out_ref[...] = (lg - mx) - jnp.log(
        jnp.sum(jnp.exp(lg - mx), axis=1, keepdims=True))


def _tc_post(deg2, acc2, y, b, m, fc_w, fc_b):
    blk = pl.BlockSpec((_TCBLK, NHID), lambda i: (i, 0))
    return pl.pallas_call(
        _tc_post_body,
        grid=(N // _TCBLK,),
        in_specs=[
            pl.BlockSpec((_NC, _TCBLK, 16), lambda i: (0, i, 0)),
            pl.BlockSpec((_NC, _TCBLK, NHID), lambda i: (0, i, 0)),
            blk,
            pl.BlockSpec((1, NHID), lambda i: (0, 0)),
            blk,
            pl.BlockSpec((NHID, NCLASS), lambda i: (0, 0)),
            pl.BlockSpec((1, NCLASS), lambda i: (0, 0)),
        ],
        out_specs=pl.BlockSpec((_TCBLK, NCLASS), lambda i: (i, 0)),
        out_shape=jax.ShapeDtypeStruct((N, NCLASS), jnp.float32),
    )(deg2, acc2, y, b, m, fc_w, fc_b)


def kernel(x, edge_index, W0, b0, W1, b1, W2, b2, W3, b3, W4, b4, W5, b5,
           fc_W, fc_b):
    Ws = [W0, W1, W2, W3, W4, W5]
    bs = [b.reshape(1, NHID) for b in (b0, b1, b2, b3, b4, b5)]
    src3 = edge_index[0].reshape(_NW, _NJ, _CH)
    dst3s = edge_index[1].reshape(_NW, _NJ, _CH)
    dst3 = edge_index[1].reshape(_NW, _DNJ, _DCH)

    deg2 = _sc_degree(dst3).reshape(_NC, N, 16)
    y = _tc_pre(deg2, x, Ws[0])
    m = None
    for i in range(NLAYER):
        acc2 = _sc_spmm(y, src3, dst3s).reshape(_NC, N, NHID)
        if i < NLAYER - 1:
            y, m = _tc_mid(deg2, acc2, y, bs[i], Ws[i + 1], m)
        else:
            out = _tc_post(deg2, acc2, y, bs[i], m, fc_W,
                           fc_b.reshape(1, NCLASS))
    return out


# speedup vs baseline: 1.2843x; 1.0164x over previous
import functools

import jax
import jax.numpy as jnp
from jax import lax
from jax.experimental import pallas as pl
from jax.experimental.pallas import tpu as pltpu
from jax.experimental.pallas import tpu_sc as plsc

N = 10000
E = 320000
NFEAT = 128
NHID = 128
NCLASS = 40
NLAYER = 6

_NC = 2
_NS = 16
_NW = _NC * _NS
_EPT = E // _NW
_CH = 125
_NJ = _EPT // _CH
_HF = _NJ // 2
_RB = 624
_RL = 640

_DCH = 100
_DNJ = _EPT // _DCH

_TCBLK = 2000


def _zero_vmem(ref, rows, cols):
    @pl.loop(0, rows)
    def _(r):
        @pl.loop(0, cols // 16)
        def _(c):
            ref[r, pl.ds(c * 16, 16)] = jnp.zeros((16,), jnp.float32)


def _sc_degree(dst3):
    mesh = plsc.VectorSubcoreMesh(core_axis_name="c", subcore_axis_name="s")

    @functools.partial(
        pl.kernel,
        out_type=jax.ShapeDtypeStruct((_NC * N, 16), jnp.float32),
        mesh=mesh,
        scratch_types=[
            pltpu.VMEM_SHARED((N, 16), jnp.float32),
            pltpu.VMEM((_DNJ, _DCH), jnp.int32),
            pltpu.VMEM((_DCH, 16), jnp.float32),
            pltpu.VMEM((128, 16), jnp.float32),
        ],
    )
    def k(dst_hbm, out_hbm, hist_sh, dstv, ones_v, zbuf):
        cid = lax.axis_index("c")
        sid = lax.axis_index("s")
        wid = cid * _NS + sid
        @pl.loop(0, _DCH)
        def _(r):
            ones_v[r, pl.ds(0, 16)] = jnp.ones((16,), jnp.float32)
        _zero_vmem(zbuf, 128, 16)
        for t in range(5):
            pltpu.sync_copy(zbuf, hist_sh.at[pl.ds(sid * _RB + t * 128, 128)])
        plsc.subcore_barrier()
        pltpu.sync_copy(dst_hbm.at[wid], dstv)
        @pl.loop(0, _DNJ)
        def _(j):
            pltpu.sync_copy(ones_v, hist_sh.at[dstv.at[j]], add=True)
        plsc.subcore_barrier()
        pltpu.sync_copy(
            hist_sh.at[pl.ds(sid * _RB, _RL)],
            out_hbm.at[pl.ds(cid * N + sid * _RB, _RL)],
        )

    return k(dst3)


def _sc_spmm(y, src3, dst3):
    mesh = plsc.VectorSubcoreMesh(core_axis_name="c", subcore_axis_name="s")

    @functools.partial(
        pl.kernel,
        out_type=jax.ShapeDtypeStruct((_NC * N, NHID), jnp.float32),
        mesh=mesh,
        scratch_types=[
            pltpu.VMEM_SHARED((N, NHID), jnp.float32),
            pltpu.VMEM((_HF, _CH), jnp.int32),
            pltpu.VMEM((_HF, _CH), jnp.int32),
            pltpu.VMEM((_CH, NHID), jnp.float32),
            pltpu.VMEM((_CH, NHID), jnp.float32),
            pltpu.SemaphoreType.DMA,
            pltpu.SemaphoreType.DMA,
        ],
    )
    def k(y_hbm, src_hbm, dst_hbm, out_hbm, acc_sh, srcv, dstv, rows0, rows1,
          sem0, sem1):
        cid = lax.axis_index("c")
        sid = lax.axis_index("s")
        wid = cid * _NS + sid
        _zero_vmem(rows0, 64, NHID)
        for t in range(10):
            pltpu.sync_copy(rows0.at[pl.ds(0, 64)],
                            acc_sh.at[pl.ds(sid * _RB + t * 64, 64)])
        plsc.subcore_barrier()
        pltpu.sync_copy(src_hbm.at[wid, pl.ds(0, _HF)], srcv)
        pltpu.sync_copy(dst_hbm.at[wid, pl.ds(0, _HF)], dstv)
        pltpu.async_copy(y_hbm.at[srcv.at[0]], rows0, sem0)
        for h in range(2):
            @pl.loop(0, _HF, step=2)
            def _(j):
                pltpu.async_copy(y_hbm.at[srcv.at[j + 1]], rows1, sem1)
                pltpu.make_async_copy(y_hbm.at[srcv.at[j]], rows0, sem0).wait()
                pltpu.sync_copy(rows0, acc_sh.at[dstv.at[j]], add=True)
                @pl.when(j + 2 < _HF)
                def _():
                    pltpu.async_copy(y_hbm.at[srcv.at[j + 2]], rows0, sem0)
                pltpu.make_async_copy(y_hbm.at[srcv.at[j + 1]], rows1,
                                      sem1).wait()
                pltpu.sync_copy(rows1, acc_sh.at[dstv.at[j + 1]], add=True)
            if h == 0:
                pltpu.sync_copy(src_hbm.at[wid, pl.ds(_HF, _HF)], srcv)
                pltpu.sync_copy(dst_hbm.at[wid, pl.ds(_HF, _HF)], dstv)
                pltpu.async_copy(y_hbm.at[srcv.at[0]], rows0, sem0)
        plsc.subcore_barrier()
        pltpu.sync_copy(
            acc_sh.at[pl.ds(sid * _RB, _RL)],
            out_hbm.at[pl.ds(cid * N + sid * _RB, _RL)],
        )

    return k(y, src3, dst3)


def _dinv_block(deg2_blk):
    deg = deg2_blk[0, :, 0] + deg2_blk[1, :, 0] + 1.0
    return lax.rsqrt(deg)[:, None]


def _tc_pre_body(deg2_ref, x_ref, w_ref, y_ref):
    dinv = _dinv_block(deg2_ref[...])
    y_ref[...] = dinv * jnp.dot(x_ref[...], w_ref[...],
                                preferred_element_type=jnp.float32)


def _tc_pre(deg2, x, w):
    return pl.pallas_call(
        _tc_pre_body,
        grid=(N // _TCBLK,),
        in_specs=[
            pl.BlockSpec((_NC, _TCBLK, 16), lambda i: (0, i, 0)),
            pl.BlockSpec((_TCBLK, NFEAT), lambda i: (i, 0)),
            pl.BlockSpec((NFEAT, NHID), lambda i: (0, 0)),
        ],
        out_specs=pl.BlockSpec((_TCBLK, NHID), lambda i: (i, 0)),
        out_shape=jax.ShapeDtypeStruct((N, NHID), jnp.float32),
    )(deg2, x, w)


def _tc_mid_body(deg2_ref, acc2_ref, y_ref, b_ref, w_ref, *rest, first):
    if first:
        ynext_ref, mout_ref = rest
    else:
        m_ref, ynext_ref, mout_ref = rest
    dinv = _dinv_block(deg2_ref[...])
    agg = acc2_ref[0] + acc2_ref[1] + y_ref[...]
    h = jnp.maximum(dinv * agg + b_ref[...], 0.0)
    m = h if first else jnp.maximum(m_ref[...], h)
    mout_ref[...] = m
    ynext_ref[...] = dinv * jnp.dot(h, w_ref[...],
                                    preferred_element_type=jnp.float32)


def _tc_mid(deg2, acc2, y, b, w, m):
    first = m is None
    blk = pl.BlockSpec((_TCBLK, NHID), lambda i: (i, 0))
    in_specs = [
        pl.BlockSpec((_NC, _TCBLK, 16), lambda i: (0, i, 0)),
        pl.BlockSpec((_NC, _TCBLK, NHID), lambda i: (0, i, 0)),
        blk,
        pl.BlockSpec((1, NHID), lambda i: (0, 0)),
        pl.BlockSpec((NHID, NHID), lambda i: (0, 0)),
    ]
    args = [deg2, acc2, y, b, w]
    if not first:
        in_specs.append(blk)
        args.append(m)
    return pl.pallas_call(
        functools.partial(_tc_mid_body, first=first),
        grid=(N // _TCBLK,),
        in_specs=in_specs,
        out_specs=[blk, blk],
        out_shape=[
            jax.ShapeDtypeStruct((N, NHID), jnp.float32),
            jax.ShapeDtypeStruct((N, NHID), jnp.float32),
        ],
    )(*args)


def _tc_post_body(deg2_ref, acc2_ref, y_ref, b_ref, m_ref, fcw_ref, fcb_ref,
                  out_ref):
    dinv = _dinv_block(deg2_ref[...])
    agg = acc2_ref[0] + acc2_ref[1] + y_ref[...]
    h = jnp.maximum(dinv * agg + b_ref[...], 0.0)
    m = jnp.maximum(m_ref[...], h)
    lg = jnp.dot(m, fcw_ref[...], preferred_element_type=jnp.float32)
    lg = lg + fcb_ref[...]
    mx = jnp.max(lg, axis=1, keepdims=True)
    out_ref[...] = (lg - mx) - jnp.log(
        jnp.sum(jnp.exp(lg - mx), axis=1, keepdims=True))


def _tc_post(deg2, acc2, y, b, m, fc_w, fc_b):
    blk = pl.BlockSpec((_TCBLK, NHID), lambda i: (i, 0))
    return pl.pallas_call(
        _tc_post_body,
        grid=(N // _TCBLK,),
        in_specs=[
            pl.BlockSpec((_NC, _TCBLK, 16), lambda i: (0, i, 0)),
            pl.BlockSpec((_NC, _TCBLK, NHID), lambda i: (0, i, 0)),
            blk,
            pl.BlockSpec((1, NHID), lambda i: (0, 0)),
            blk,
            pl.BlockSpec((NHID, NCLASS), lambda i: (0, 0)),
            pl.BlockSpec((1, NCLASS), lambda i: (0, 0)),
        ],
        out_specs=pl.BlockSpec((_TCBLK, NCLASS), lambda i: (i, 0)),
        out_shape=jax.ShapeDtypeStruct((N, NCLASS), jnp.float32),
    )(deg2, acc2, y, b, m, fc_w, fc_b)


def kernel(x, edge_index, W0, b0, W1, b1, W2, b2, W3, b3, W4, b4, W5, b5,
           fc_W, fc_b):
    Ws = [W0, W1, W2, W3, W4, W5]
    bs = [b.reshape(1, NHID) for b in (b0, b1, b2, b3, b4, b5)]
    src3 = edge_index[0].reshape(_NW, _NJ, _CH)
    dst3s = edge_index[1].reshape(_NW, _NJ, _CH)
    dst3 = edge_index[1].reshape(_NW, _DNJ, _DCH)

    deg2 = _sc_degree(dst3).reshape(_NC, N, 16)
    y = _tc_pre(deg2, x, Ws[0])
    m = None
    for i in range(NLAYER):
        acc2 = _sc_spmm(y, src3, dst3s).reshape(_NC, N, NHID)
        if i < NLAYER - 1:
            y, m = _tc_mid(deg2, acc2, y, bs[i], Ws[i + 1], m)
        else:
            out = _tc_post(deg2, acc2, y, bs[i], m, fc_W,
                           fc_b.reshape(1, NCLASS))
    return out
